# baseline (speedup 1.0000x reference)
import functools
import jax
import jax.numpy as jnp
from jax import lax
from jax.experimental import pallas as pl
from jax.experimental.pallas import tpu as pltpu
from jax.experimental.pallas import tpu_sc as plsc

_SS = ((128, 128), (64, 64), (32, 32), (16, 16))
_BASES = (0, 16384, 20480, 21504)
_NV = 21760
_H = 8
_P = 4
_L = 4
_D = 32
_HLP = _H * _L * _P

_NW = 32
_QI = 5
_ROWS = _QI * 4 * 128


def _value_proj(value2, Wv, bv):
    n, C = value2.shape
    R = 512
    grid = (n // R,)

    def body(v_ref, w_ref, b_ref, o_ref):
        o_ref[...] = jnp.dot(v_ref[...], w_ref[...],
                             preferred_element_type=jnp.float32) + b_ref[...]

    return pl.pallas_call(
        body,
        grid=grid,
        in_specs=[
            pl.BlockSpec((R, C), lambda i: (i, 0)),
            pl.BlockSpec((C, C), lambda i: (0, 0)),
            pl.BlockSpec((1, C), lambda i: (0, 0)),
        ],
        out_specs=pl.BlockSpec((R, C), lambda i: (i, 0)),
        out_shape=jax.ShapeDtypeStruct((n, C), jnp.float32),
    )(value2, Wv, bv.reshape(1, C))


def _out_proj(x2, Wout, bout):
    n, C = x2.shape
    R = 2000
    grid = (n // R,)

    def body(x_ref, w_ref, b_ref, o_ref):
        o_ref[...] = jnp.dot(x_ref[...], w_ref[...],
                             preferred_element_type=jnp.float32) + b_ref[...]

    return pl.pallas_call(
        body,
        grid=grid,
        in_specs=[
            pl.BlockSpec((R, C), lambda i: (i, 0)),
            pl.BlockSpec((C, C), lambda i: (0, 0)),
            pl.BlockSpec((1, C), lambda i: (0, 0)),
        ],
        out_specs=pl.BlockSpec((R, C), lambda i: (i, 0)),
        out_shape=jax.ShapeDtypeStruct((n, C), jnp.float32),
    )(x2, Wout, bout.reshape(1, C))


def _prep(q2, rp2, Wox, Woy, box, boy, Wa, ba, nq):
    n, C = q2.shape
    B = 2000
    blocks_per_b = nq // B
    grid = (n // B,)

    def body(q_ref, rp_ref, wox_ref, woy_ref, box_ref, boy_ref, wa_ref,
             ba_ref, idx_ref, wgt_ref):
        pid = pl.program_id(0)
        b = pid // blocks_per_b

        q = q_ref[...]
        off_x = jnp.dot(q, wox_ref[...],
                        preferred_element_type=jnp.float32) + box_ref[...]
        off_y = jnp.dot(q, woy_ref[...],
                        preferred_element_type=jnp.float32) + boy_ref[...]
        logits = jnp.dot(q, wa_ref[...],
                         preferred_element_type=jnp.float32) + ba_ref[...]

        segs = []
        for h in range(_H):
            seg = logits[:, h * 16:(h + 1) * 16]
            m = jnp.max(seg, axis=1, keepdims=True)
            e = jnp.exp(seg - m)
            segs.append(e / jnp.sum(e, axis=1, keepdims=True))
        aw = jnp.concatenate(segs, axis=1)

        lane = lax.broadcasted_iota(jnp.int32, (B, _HLP), 1)
        l_vec = (lane >> 2) & 3
        h_vec = lane >> 4
        wl_i = jnp.where(l_vec == 0, 128,
                         jnp.where(l_vec == 1, 64,
                                   jnp.where(l_vec == 2, 32, 16)))
        base_i = jnp.where(l_vec == 0, _BASES[0],
                           jnp.where(l_vec == 1, _BASES[1],
                                     jnp.where(l_vec == 2, _BASES[2],
                                               _BASES[3])))
        wl_f = wl_i.astype(jnp.float32)

        rp_x = rp_ref[:, 0:1]
        rp_y = rp_ref[:, 1:2]
        x = (rp_x + off_x / wl_f) * wl_f - 0.5
        y = (rp_y + off_y / wl_f) * wl_f - 0.5

        x0f = jnp.floor(x)
        y0f = jnp.floor(y)
        x0 = x0f.astype(jnp.int32)
        y0 = y0f.astype(jnp.int32)
        wx1 = x - x0f
        wy1 = y - y0f
        wx0 = 1.0 - wx1
        wy0 = 1.0 - wy1
        x1 = x0 + 1
        y1 = y0 + 1

        def side(c):
            v = ((c >= 0) & (c < wl_i)).astype(jnp.float32)
            cc = jnp.clip(c, 0, wl_i - 1)
            return v, cc

        vx0, x0c = side(x0)
        vx1, x1c = side(x1)
        vy0, y0c = side(y0)
        vy1, y1c = side(y1)

        boff = b * (_NV * _H)
        taps = ((y0c, x0c, vy0 * vx0 * wy0 * wx0),
                (y0c, x1c, vy0 * vx1 * wy0 * wx1),
                (y1c, x0c, vy1 * vx0 * wy1 * wx0),
                (y1c, x1c, vy1 * vx1 * wy1 * wx1))
        for t, (yc, xc, w) in enumerate(taps):
            idx_ref[:, t, :] = (base_i + yc * wl_i + xc) * _H + h_vec + boff
            wgt_ref[:, t, :] = aw * w

    return pl.pallas_call(
        body,
        grid=grid,
        in_specs=[
            pl.BlockSpec((B, C), lambda i: (i, 0)),
            pl.BlockSpec((B, 2), lambda i: (i, 0)),
            pl.BlockSpec((C, _HLP), lambda i: (0, 0)),
            pl.BlockSpec((C, _HLP), lambda i: (0, 0)),
            pl.BlockSpec((1, _HLP), lambda i: (0, 0)),
            pl.BlockSpec((1, _HLP), lambda i: (0, 0)),
            pl.BlockSpec((C, _HLP), lambda i: (0, 0)),
            pl.BlockSpec((1, _HLP), lambda i: (0, 0)),
        ],
        out_specs=[
            pl.BlockSpec((B, 4, _HLP), lambda i: (i, 0, 0)),
            pl.BlockSpec((B, 4, _HLP), lambda i: (i, 0, 0)),
        ],
        out_shape=[
            jax.ShapeDtypeStruct((n, 4, _HLP), jnp.int32),
            jax.ShapeDtypeStruct((n, 4, _HLP), jnp.float32),
        ],
    )(q2, rp2, Wox, Woy, box, boy, Wa, ba)


def _make_sc_gather(n_items):
    per_w = n_items // _NW
    iters = per_w // _QI
    assert per_w % _QI == 0

    mesh = plsc.VectorSubcoreMesh(core_axis_name="c", subcore_axis_name="s")

    @functools.partial(
        pl.kernel,
        mesh=mesh,
        compiler_params=pltpu.CompilerParams(use_tc_tiling_on_sc=False),
        out_type=jax.ShapeDtypeStruct((n_items, 2, _HLP), jnp.float32),
        scratch_types=[
            pltpu.VMEM((_QI, 4, _HLP), jnp.int32),
            pltpu.VMEM((_QI, 4, _HLP), jnp.float32),
            pltpu.VMEM((_ROWS, _D), jnp.float32),
            pltpu.VMEM((_QI, 2, _HLP), jnp.float32),
            [pltpu.SemaphoreType.DMA] * _QI,
        ],
    )
    def sc_gather(table_hbm, idx_hbm, wgt_hbm, out_hbm,
                  idx_v, wgt_v, rows_v, out_v, sems):
        wid = lax.axis_index("s") * 2 + lax.axis_index("c")

        def it_body(it, carry):
            n0 = wid * per_w + it * _QI
            pltpu.sync_copy(idx_hbm.at[pl.ds(n0, _QI)], idx_v)
            pltpu.sync_copy(wgt_hbm.at[pl.ds(n0, _QI)], wgt_v)
            copies = []
            for i in range(_QI):
                for tap in range(4):
                    copies.append(pltpu.async_copy(
                        table_hbm.at[idx_v.at[i, tap]],
                        rows_v.at[pl.ds((i * 4 + tap) * _HLP, _HLP)],
                        sems[i]))

            def item_body(i, c2):
                for k in range(_QI):
                    @pl.when(i == k)
                    def _wait():
                        for c in copies[4 * k:4 * k + 4]:
                            c.wait()
                accs = [jnp.zeros((16,), jnp.float32) for _ in range(2 * _H)]
                rb = i * 4 * _HLP
                for tap in range(4):
                    for h in range(_H):
                        wv = wgt_v[i, tap, pl.ds(h * 16, 16)]
                        for k in range(16):
                            w = wv[k]
                            row = rb + tap * _HLP + h * 16 + k
                            accs[2 * h] = accs[2 * h] + w * rows_v[row, pl.ds(0, 16)]
                            accs[2 * h + 1] = (accs[2 * h + 1]
                                               + w * rows_v[row, pl.ds(16, 16)])
                for h in range(_H):
                    out_v[i, h // 4, pl.ds((h % 4) * 32, 16)] = accs[2 * h]
                    out_v[i, h // 4, pl.ds((h % 4) * 32 + 16, 16)] = accs[2 * h + 1]
                return c2

            lax.fori_loop(0, _QI, item_body, 0)
            pltpu.sync_copy(out_v, out_hbm.at[pl.ds(n0, _QI)])
            return carry

        lax.fori_loop(0, iters, it_body, 0)

    return sc_gather


def kernel(query, value, reference_points, spatial_shapes, Wv, bv, Wo, bo,
           Wa, ba, Wout, bout):
    bs, nq, C = query.shape
    n_items = bs * nq

    table = _value_proj(value.reshape(bs * _NV, C), Wv, bv)
    table = table.reshape(bs * _NV * _H, _D)

    Wo6 = Wo.reshape(C, _H, _L, _P, 2)
    bo6 = bo.reshape(_H, _L, _P, 2)
    idx, wgt = _prep(
        query.reshape(n_items, C),
        reference_points.reshape(n_items, 2),
        Wo6[..., 0].reshape(C, _HLP),
        Wo6[..., 1].reshape(C, _HLP),
        bo6[..., 0].reshape(1, _HLP),
        bo6[..., 1].reshape(1, _HLP),
        Wa,
        ba.reshape(1, _HLP),
        nq)

    out = _make_sc_gather(n_items)(table, idx, wgt)
    res = _out_proj(out.reshape(n_items, C), Wout, bout)
    return res.reshape(bs, nq, C)


# speedup vs baseline: 1.8736x; 1.8736x over previous
import functools
import jax
import jax.numpy as jnp
from jax import lax
from jax.experimental import pallas as pl
from jax.experimental.pallas import tpu as pltpu
from jax.experimental.pallas import tpu_sc as plsc

_SS = ((128, 128), (64, 64), (32, 32), (16, 16))
_BASES = (0, 16384, 20480, 21504)
_NV = 21760
_H = 8
_P = 4
_L = 4
_D = 32
_HLP = _H * _L * _P

_NW = 32
_QI = 5
_ROWS = _QI * 4 * 128


def _value_proj(value2, Wv, bv):
    n, C = value2.shape
    R = 512
    grid = (n // R,)

    def body(v_ref, w_ref, b_ref, o_ref):
        o_ref[...] = jnp.dot(v_ref[...], w_ref[...],
                             preferred_element_type=jnp.float32) + b_ref[...]

    return pl.pallas_call(
        body,
        grid=grid,
        in_specs=[
            pl.BlockSpec((R, C), lambda i: (i, 0)),
            pl.BlockSpec((C, C), lambda i: (0, 0)),
            pl.BlockSpec((1, C), lambda i: (0, 0)),
        ],
        out_specs=pl.BlockSpec((R, C), lambda i: (i, 0)),
        out_shape=jax.ShapeDtypeStruct((n, C), jnp.float32),
    )(value2, Wv, bv.reshape(1, C))


def _out_proj(x2, Wout, bout):
    n, C = x2.shape
    R = 2000
    grid = (n // R,)

    def body(x_ref, w_ref, b_ref, o_ref):
        o_ref[...] = jnp.dot(x_ref[...], w_ref[...],
                             preferred_element_type=jnp.float32) + b_ref[...]

    return pl.pallas_call(
        body,
        grid=grid,
        in_specs=[
            pl.BlockSpec((R, C), lambda i: (i, 0)),
            pl.BlockSpec((C, C), lambda i: (0, 0)),
            pl.BlockSpec((1, C), lambda i: (0, 0)),
        ],
        out_specs=pl.BlockSpec((R, C), lambda i: (i, 0)),
        out_shape=jax.ShapeDtypeStruct((n, C), jnp.float32),
    )(x2, Wout, bout.reshape(1, C))


def _prep(q2, rp2, Wox, Woy, box, boy, Wa, ba, nq):
    n, C = q2.shape
    B = 2000
    blocks_per_b = nq // B
    grid = (n // B,)

    def body(q_ref, rp_ref, wox_ref, woy_ref, box_ref, boy_ref, wa_ref,
             ba_ref, idx_ref, wgt_ref):
        pid = pl.program_id(0)
        b = pid // blocks_per_b

        q = q_ref[...]
        off_x = jnp.dot(q, wox_ref[...],
                        preferred_element_type=jnp.float32) + box_ref[...]
        off_y = jnp.dot(q, woy_ref[...],
                        preferred_element_type=jnp.float32) + boy_ref[...]
        logits = jnp.dot(q, wa_ref[...],
                         preferred_element_type=jnp.float32) + ba_ref[...]

        segs = []
        for h in range(_H):
            seg = logits[:, h * 16:(h + 1) * 16]
            m = jnp.max(seg, axis=1, keepdims=True)
            e = jnp.exp(seg - m)
            segs.append(e / jnp.sum(e, axis=1, keepdims=True))
        aw = jnp.concatenate(segs, axis=1)

        lane = lax.broadcasted_iota(jnp.int32, (B, _HLP), 1)
        l_vec = (lane >> 2) & 3
        h_vec = lane >> 4
        wl_i = jnp.where(l_vec == 0, 128,
                         jnp.where(l_vec == 1, 64,
                                   jnp.where(l_vec == 2, 32, 16)))
        base_i = jnp.where(l_vec == 0, _BASES[0],
                           jnp.where(l_vec == 1, _BASES[1],
                                     jnp.where(l_vec == 2, _BASES[2],
                                               _BASES[3])))
        wl_f = wl_i.astype(jnp.float32)

        rp_x = rp_ref[:, 0:1]
        rp_y = rp_ref[:, 1:2]
        x = (rp_x + off_x / wl_f) * wl_f - 0.5
        y = (rp_y + off_y / wl_f) * wl_f - 0.5

        x0f = jnp.floor(x)
        y0f = jnp.floor(y)
        x0 = x0f.astype(jnp.int32)
        y0 = y0f.astype(jnp.int32)
        wx1 = x - x0f
        wy1 = y - y0f
        wx0 = 1.0 - wx1
        wy0 = 1.0 - wy1
        x1 = x0 + 1
        y1 = y0 + 1

        def side(c):
            v = ((c >= 0) & (c < wl_i)).astype(jnp.float32)
            cc = jnp.clip(c, 0, wl_i - 1)
            return v, cc

        vx0, x0c = side(x0)
        vx1, x1c = side(x1)
        vy0, y0c = side(y0)
        vy1, y1c = side(y1)

        boff = b * (_NV * _H)
        taps = ((y0c, x0c, vy0 * vx0 * wy0 * wx0),
                (y0c, x1c, vy0 * vx1 * wy0 * wx1),
                (y1c, x0c, vy1 * vx0 * wy1 * wx0),
                (y1c, x1c, vy1 * vx1 * wy1 * wx1))
        for t, (yc, xc, w) in enumerate(taps):
            idx_ref[:, t, :] = (base_i + yc * wl_i + xc) * _H + h_vec + boff
            wgt_ref[:, t, :] = aw * w

    return pl.pallas_call(
        body,
        grid=grid,
        in_specs=[
            pl.BlockSpec((B, C), lambda i: (i, 0)),
            pl.BlockSpec((B, 2), lambda i: (i, 0)),
            pl.BlockSpec((C, _HLP), lambda i: (0, 0)),
            pl.BlockSpec((C, _HLP), lambda i: (0, 0)),
            pl.BlockSpec((1, _HLP), lambda i: (0, 0)),
            pl.BlockSpec((1, _HLP), lambda i: (0, 0)),
            pl.BlockSpec((C, _HLP), lambda i: (0, 0)),
            pl.BlockSpec((1, _HLP), lambda i: (0, 0)),
        ],
        out_specs=[
            pl.BlockSpec((B, 4, _HLP), lambda i: (i, 0, 0)),
            pl.BlockSpec((B, 4, _HLP), lambda i: (i, 0, 0)),
        ],
        out_shape=[
            jax.ShapeDtypeStruct((n, 4, _HLP), jnp.int32),
            jax.ShapeDtypeStruct((n, 4, _HLP), jnp.float32),
        ],
    )(q2, rp2, Wox, Woy, box, boy, Wa, ba)


def _make_sc_gather(n_items):
    per_w = n_items // _NW
    iters = per_w // _QI
    assert per_w % _QI == 0

    mesh = plsc.VectorSubcoreMesh(core_axis_name="c", subcore_axis_name="s")

    @functools.partial(
        pl.kernel,
        mesh=mesh,
        compiler_params=pltpu.CompilerParams(use_tc_tiling_on_sc=False),
        out_type=jax.ShapeDtypeStruct((n_items, 2, _HLP), jnp.float32),
        scratch_types=[
            pltpu.VMEM((_QI, 4, _HLP), jnp.int32),
            pltpu.VMEM((_QI, 4, _HLP), jnp.float32),
            pltpu.VMEM((_ROWS, _D), jnp.float32),
            pltpu.VMEM((_QI, 2, _HLP), jnp.float32),
            [pltpu.SemaphoreType.DMA] * _QI,
        ],
    )
    def sc_gather(table_hbm, idx_hbm, wgt_hbm, out_hbm,
                  idx_v, wgt_v, rows_v, out_v, sems):
        wid = lax.axis_index("s") * 2 + lax.axis_index("c")

        def it_body(it, carry):
            n0 = wid * per_w + it * _QI
            pltpu.sync_copy(idx_hbm.at[pl.ds(n0, _QI)], idx_v)
            pltpu.sync_copy(wgt_hbm.at[pl.ds(n0, _QI)], wgt_v)
            copies = []
            for i in range(_QI):
                for tap in range(4):
                    copies.append(pltpu.async_copy(
                        table_hbm.at[idx_v.at[i, tap]],
                        rows_v.at[pl.ds((i * 4 + tap) * _HLP, _HLP)],
                        sems[i]))

            def item_body(i, c2):
                for k in range(_QI):
                    @pl.when(i == k)
                    def _wait():
                        for c in copies[4 * k:4 * k + 4]:
                            c.wait()
                rb = i * 4 * _HLP
                for h in range(_H):
                    z = jnp.zeros((16,), jnp.float32)
                    a0, a1, b0, b1 = z, z, z, z
                    for tap in range(4):
                        wv = wgt_v[i, tap, pl.ds(h * 16, 16)]
                        for k in range(16):
                            w = wv[k]
                            row = rb + tap * _HLP + h * 16 + k
                            if k % 2 == 0:
                                a0 = a0 + w * rows_v[row, pl.ds(0, 16)]
                                a1 = a1 + w * rows_v[row, pl.ds(16, 16)]
                            else:
                                b0 = b0 + w * rows_v[row, pl.ds(0, 16)]
                                b1 = b1 + w * rows_v[row, pl.ds(16, 16)]
                    out_v[i, h // 4, pl.ds((h % 4) * 32, 16)] = a0 + b0
                    out_v[i, h // 4, pl.ds((h % 4) * 32 + 16, 16)] = a1 + b1
                return c2

            lax.fori_loop(0, _QI, item_body, 0)
            pltpu.sync_copy(out_v, out_hbm.at[pl.ds(n0, _QI)])
            return carry

        lax.fori_loop(0, iters, it_body, 0)

    return sc_gather


def kernel(query, value, reference_points, spatial_shapes, Wv, bv, Wo, bo,
           Wa, ba, Wout, bout):
    bs, nq, C = query.shape
    n_items = bs * nq

    table = _value_proj(value.reshape(bs * _NV, C), Wv, bv)
    table = table.reshape(bs * _NV * _H, _D)

    Wo6 = Wo.reshape(C, _H, _L, _P, 2)
    bo6 = bo.reshape(_H, _L, _P, 2)
    idx, wgt = _prep(
        query.reshape(n_items, C),
        reference_points.reshape(n_items, 2),
        Wo6[..., 0].reshape(C, _HLP),
        Wo6[..., 1].reshape(C, _HLP),
        bo6[..., 0].reshape(1, _HLP),
        bo6[..., 1].reshape(1, _HLP),
        Wa,
        ba.reshape(1, _HLP),
        nq)

    out = _make_sc_gather(n_items)(table, idx, wgt)
    res = _out_proj(out.reshape(n_items, C), Wout, bout)
    return res.reshape(bs, nq, C)
